# Initial kernel scaffold; baseline (speedup 1.0000x reference)
import jax
import jax.numpy as jnp
from jax.experimental import pallas as pl


def kernel(x, edge_index, batch, W1, b1, W2, b2, eps, Wn, bn, Wp, bp):
    raise NotImplementedError("write your pallas kernel here")


# speedup vs baseline: 7.2894x; 7.2894x over previous
import functools

import jax
import jax.numpy as jnp
from jax import lax
from jax.experimental import pallas as pl
from jax.experimental.pallas import tpu as pltpu
from jax.experimental.pallas import tpu_sc as plsc

N = 10000
E = 320000
D = 128
G = 64

NC = 2
NS = 16
NW = NC * NS
EPW = E // NW
CH = 80
NCHUNK = EPW // CH
NP = 10240
RPT = NP // NS
ZR = 32

_sc_mesh = plsc.VectorSubcoreMesh(core_axis_name="c", subcore_axis_name="s")


@functools.partial(
    pl.kernel,
    out_type=jax.ShapeDtypeStruct((NC, NP, D), jnp.float32),
    mesh=_sc_mesh,
    scratch_types=[
        pltpu.VMEM((NCHUNK, CH), jnp.int32),
        pltpu.VMEM((NCHUNK, CH), jnp.int32),
        pltpu.VMEM((CH, D), jnp.float32),
        pltpu.VMEM((ZR, D), jnp.float32),
        pltpu.VMEM_SHARED((NP, D), jnp.float32),
        pltpu.SemaphoreType.DMA,
    ],
)
def _sc_aggregate(h_hbm, src_hbm, dst_hbm, out_hbm, sidx, didx, rows, zbuf, agg_sh, sem):
    c = lax.axis_index("c")
    s = lax.axis_index("s")
    w = c * NS + s

    zeros16 = jnp.zeros((16,), jnp.float32)

    def _zrow(i, carry):
        for j in range(D // 16):
            zbuf[i, pl.ds(j * 16, 16)] = zeros16
        return carry

    lax.fori_loop(0, ZR, _zrow, 0)

    def _zcopy(r, carry):
        pltpu.sync_copy(zbuf, agg_sh.at[pl.ds(s * RPT + r * ZR, ZR)])
        return carry

    lax.fori_loop(0, RPT // ZR, _zcopy, 0)

    pltpu.sync_copy(src_hbm.at[w], sidx)
    pltpu.sync_copy(dst_hbm.at[w], didx)

    plsc.subcore_barrier()

    def _chunk(g, carry):
        pltpu.async_copy(h_hbm.at[sidx.at[g]], rows, sem).wait()
        pltpu.sync_copy(rows, agg_sh.at[didx.at[g]], add=True)
        return carry

    lax.fori_loop(0, NCHUNK, _chunk, 0)

    plsc.subcore_barrier()

    pltpu.sync_copy(agg_sh.at[pl.ds(s * RPT, RPT)], out_hbm.at[c, pl.ds(s * RPT, RPT)])


R_MLP = 2000


def _mlp_body(scale_ref, h_ref, a0_ref, a1_ref, w1_ref, b1_ref, w2_ref, b2_ref, o_ref):
    z = h_ref[...] * scale_ref[0, 0] + a0_ref[0] + a1_ref[0]
    z = jnp.dot(z, w1_ref[...], preferred_element_type=jnp.float32) + b1_ref[...]
    z = jnp.maximum(z, 0.0)
    z = jnp.dot(z, w2_ref[...], preferred_element_type=jnp.float32) + b2_ref[...]
    o_ref[...] = jnp.maximum(z, 0.0)


def _tc_mlp(scale, h, agg, W1i, b1i, W2i, b2i):
    return pl.pallas_call(
        _mlp_body,
        grid=(N // R_MLP,),
        in_specs=[
            pl.BlockSpec((1, 1), lambda i: (0, 0)),
            pl.BlockSpec((R_MLP, D), lambda i: (i, 0)),
            pl.BlockSpec((1, R_MLP, D), lambda i: (0, i, 0)),
            pl.BlockSpec((1, R_MLP, D), lambda i: (1, i, 0)),
            pl.BlockSpec((D, D), lambda i: (0, 0)),
            pl.BlockSpec((1, D), lambda i: (0, 0)),
            pl.BlockSpec((D, D), lambda i: (0, 0)),
            pl.BlockSpec((1, D), lambda i: (0, 0)),
        ],
        out_specs=pl.BlockSpec((R_MLP, D), lambda i: (i, 0)),
        out_shape=jax.ShapeDtypeStruct((N, D), jnp.float32),
    )(scale, h, agg, agg, W1i, b1i, W2i, b2i)


R_HEAD = 1000


def _head_body(h_ref, batch_ref, wn_ref, bn_ref, wp_ref, bp_ref, o_ref, acc_ref):
    i = pl.program_id(0)
    hn = jnp.dot(h_ref[...], wn_ref[...], preferred_element_type=jnp.float32) + bn_ref[...]
    hn = jnp.maximum(hn, 0.0)
    onehot = (batch_ref[...] == lax.broadcasted_iota(jnp.int32, (1, G), 1)).astype(jnp.float32)
    hn_ext = jnp.concatenate([hn, jnp.ones_like(hn)], axis=1)
    blk = lax.dot_general(onehot, hn_ext, (((0,), (0,)), ((), ())))

    @pl.when(i == 0)
    def _():
        acc_ref[...] = blk

    @pl.when(i > 0)
    def _():
        acc_ref[...] = acc_ref[...] + blk

    @pl.when(i == (N // R_HEAD) - 1)
    def _():
        sums = acc_ref[:, :D]
        cnts = acc_ref[:, D:]
        h_graph = sums / jnp.maximum(cnts, 1.0)
        o_ref[...] = jnp.dot(h_graph, wp_ref[...], preferred_element_type=jnp.float32) + bp_ref[...]


def _tc_head(h, batch2d, Wn, bn, Wp_pad, bp_pad):
    return pl.pallas_call(
        _head_body,
        grid=(N // R_HEAD,),
        in_specs=[
            pl.BlockSpec((R_HEAD, D), lambda i: (i, 0)),
            pl.BlockSpec((R_HEAD, 1), lambda i: (i, 0)),
            pl.BlockSpec((D, D), lambda i: (0, 0)),
            pl.BlockSpec((1, D), lambda i: (0, 0)),
            pl.BlockSpec((D, D), lambda i: (0, 0)),
            pl.BlockSpec((1, D), lambda i: (0, 0)),
        ],
        out_specs=pl.BlockSpec((G, D), lambda i: (0, 0)),
        out_shape=jax.ShapeDtypeStruct((G, D), jnp.float32),
        scratch_shapes=[pltpu.VMEM((G, 2 * D), jnp.float32)],
    )(h, batch2d, Wn, bn, Wp_pad, bp_pad)


def kernel(x, edge_index, batch, W1, b1, W2, b2, eps, Wn, bn, Wp, bp):
    L = W1.shape[0]
    T = Wp.shape[1]
    src = edge_index[0].astype(jnp.int32).reshape(NW, NCHUNK, CH)
    dst = edge_index[1].astype(jnp.int32).reshape(NW, NCHUNK, CH)
    batch2d = batch.astype(jnp.int32).reshape(N, 1)

    h = x
    for i in range(L):
        agg = _sc_aggregate(h, src, dst)
        scale = (1.0 + eps[i]).reshape(1, 1)
        h = _tc_mlp(scale, h, agg, W1[i], b1[i].reshape(1, D), W2[i], b2[i].reshape(1, D))

    Wp_pad = jnp.zeros((D, D), jnp.float32).at[:, :T].set(Wp)
    bp_pad = jnp.zeros((1, D), jnp.float32).at[0, :T].set(bp)
    out = _tc_head(h, batch2d, Wn, bn.reshape(1, D), Wp_pad, bp_pad)
    return out[:, :T]
